# baseline (speedup 1.0000x reference)
import functools

import jax
import jax.numpy as jnp
from jax.experimental import pallas as pl
from jax.experimental.pallas import tpu as pltpu


def _copy_body(n, half, x_ref, o_ref, vmem, load_sems, store_sems):
    loads = []
    for i in range(n):
        c = pltpu.make_async_copy(
            x_ref.at[pl.ds(i * half, half), :],
            vmem.at[pl.ds(i * half, half), :],
            load_sems.at[i],
        )
        c.start()
        loads.append(c)
    stores = []
    for i in range(n):
        loads[i].wait()
        c = pltpu.make_async_copy(
            vmem.at[pl.ds(i * half, half), :],
            o_ref.at[pl.ds(i * half, half), :],
            store_sems.at[i],
        )
        c.start()
        stores.append(c)
    for c in stores:
        c.wait()


def kernel(x):
    b, s, d = x.shape
    rows = b * s
    n = 2 if rows % 2 == 0 else 1
    half = rows // n
    x2 = x.reshape(rows, d)
    out = pl.pallas_call(
        functools.partial(_copy_body, n, half),
        in_specs=[pl.BlockSpec(memory_space=pl.ANY)],
        out_specs=pl.BlockSpec(memory_space=pl.ANY),
        scratch_shapes=[
            pltpu.VMEM((rows, d), x.dtype),
            pltpu.SemaphoreType.DMA((n,)),
            pltpu.SemaphoreType.DMA((n,)),
        ],
        out_shape=jax.ShapeDtypeStruct((rows, d), x.dtype),
    )(x2)
    return out.reshape(b, s, d)
